# Initial kernel scaffold; baseline (speedup 1.0000x reference)
import functools

import jax
import jax.numpy as jnp
from jax import lax
from jax.experimental import pallas as pl
from jax.experimental.pallas import tpu as pltpu
from jax.experimental.pallas import tpu_sc as plsc

N = 10000
E = 320000
D = 128
NW = 32
ROWS_W = 320
N_PAD = NW * ROWS_W
CHUNK = 128
EPAD = 256

_sc_mesh = plsc.VectorSubcoreMesh(core_axis_name="c", subcore_axis_name="s")


@functools.partial(
    pl.kernel,
    out_type=jax.ShapeDtypeStruct((N_PAD, D), jnp.float32),
    mesh=_sc_mesh,
    scratch_types=[
        pltpu.VMEM((CHUNK,), jnp.int32),
        pltpu.VMEM((CHUNK,), jnp.int32),
        pltpu.VMEM((CHUNK, D), jnp.float32),
        pltpu.VMEM((ROWS_W, D), jnp.float32),
        pltpu.VMEM((40,), jnp.int32),
        pltpu.SemaphoreType.DMA,
    ],
)
def _segmax(h_hbm, src_hbm, dst_hbm, est_hbm, out_hbm,
            idx_v, dst_v, rows_v, acc_v, est_v, sem):
    wid = lax.axis_index("s") * 2 + lax.axis_index("c")
    lo = wid * ROWS_W

    pltpu.sync_copy(est_hbm, est_v)
    e0 = est_v[wid]
    e1 = est_v[wid + 1] + 8

    neg = jnp.full((16,), -jnp.inf, jnp.float32)

    def init_row(r, carry):
        for cc in range(D // 16):
            acc_v[r, pl.ds(cc * 16, 16)] = neg
        return carry
    lax.fori_loop(0, ROWS_W, init_row, 0)

    nchunks = (e1 - e0 + CHUNK - 1) // CHUNK

    def chunk_body(g, carry):
        off = e0 + g * CHUNK
        pltpu.sync_copy(src_hbm.at[pl.ds(off, CHUNK)], idx_v)
        pltpu.sync_copy(dst_hbm.at[pl.ds(off, CHUNK)], dst_v)
        pltpu.async_copy(h_hbm.at[idx_v], rows_v, sem).wait()

        def edge_body(e, c2):
            d = dst_v[e] - lo

            @pl.when((d >= 0) & (d < ROWS_W))
            def _():
                for cc in range(D // 16):
                    sl = pl.ds(cc * 16, 16)
                    acc_v[d, sl] = jnp.maximum(acc_v[d, sl], rows_v[e, sl])
            return c2
        lax.fori_loop(0, CHUNK, edge_body, 0)
        return carry
    lax.fori_loop(0, nchunks, chunk_body, 0)

    def fin_row(r, carry):
        for cc in range(D // 16):
            sl = pl.ds(cc * 16, 16)
            v = acc_v[r, sl]
            acc_v[r, sl] = jnp.where(v == -jnp.inf,
                                     jnp.zeros((16,), jnp.float32), v)
        return carry
    lax.fori_loop(0, ROWS_W, fin_row, 0)

    pltpu.sync_copy(acc_v, out_hbm.at[pl.ds(lo, ROWS_W)])


BLK = 1000


def _lin_body(agg_ref, h_ref, wr_ref, wt_ref, b_ref, o_ref):
    dn = (((1,), (1,)), ((), ()))
    o_ref[...] = jax.nn.relu(
        lax.dot_general(agg_ref[...], wr_ref[...], dn,
                        preferred_element_type=jnp.float32)
        + lax.dot_general(h_ref[...], wt_ref[...], dn,
                          preferred_element_type=jnp.float32)
        + b_ref[...])


_lin = pl.pallas_call(
    _lin_body,
    grid=(N // BLK,),
    in_specs=[
        pl.BlockSpec((BLK, D), lambda i: (i, 0)),
        pl.BlockSpec((BLK, D), lambda i: (i, 0)),
        pl.BlockSpec((D, D), lambda i: (0, 0)),
        pl.BlockSpec((D, D), lambda i: (0, 0)),
        pl.BlockSpec((1, D), lambda i: (0, 0)),
    ],
    out_specs=pl.BlockSpec((BLK, D), lambda i: (i, 0)),
    out_shape=jax.ShapeDtypeStruct((N, D), jnp.float32),
)


def kernel(x, edge_index, W1_rel, b1_rel, W1_root, W_rel, b_rel, W_root):
    src = edge_index[0].astype(jnp.int32)
    dst = edge_index[1].astype(jnp.int32)

    order = jnp.argsort(dst)
    dst_s = jnp.concatenate(
        [dst[order], jnp.full((EPAD,), N_PAD, jnp.int32)])
    src_s = jnp.concatenate([src[order], jnp.zeros((EPAD,), jnp.int32)])
    bounds = jnp.searchsorted(
        dst_s[:E], (jnp.arange(NW + 1) * ROWS_W).astype(jnp.int32)
    ).astype(jnp.int32)
    est = jnp.concatenate(
        [(bounds // 8) * 8, jnp.zeros((40 - (NW + 1),), jnp.int32)])

    h = x
    for wr, b, wt in ((W1_rel, b1_rel, W1_root),
                      (W_rel, b_rel, W_root),
                      (W_rel, b_rel, W_root)):
        agg = _segmax(h, src_s, dst_s, est)
        h = _lin(agg, h, wr, wt, b.reshape(1, D))
    return h


# speedup vs baseline: 2.4408x; 2.4408x over previous
import functools

import jax
import jax.numpy as jnp
from jax import lax
from jax.experimental import pallas as pl
from jax.experimental.pallas import tpu as pltpu
from jax.experimental.pallas import tpu_sc as plsc

N = 10000
E = 320000
D = 128
NW = 32
ROWS_W = 320
N_PAD = NW * ROWS_W
CHUNK = 128
EPAD = 256

_sc_mesh = plsc.VectorSubcoreMesh(core_axis_name="c", subcore_axis_name="s")


@functools.partial(
    pl.kernel,
    out_type=jax.ShapeDtypeStruct((N_PAD, D), jnp.float32),
    mesh=_sc_mesh,
    scratch_types=[
        pltpu.VMEM((CHUNK,), jnp.int32),
        pltpu.VMEM((CHUNK,), jnp.int32),
        pltpu.VMEM((CHUNK, D), jnp.float32),
        pltpu.VMEM((ROWS_W + 1, D), jnp.float32),
        pltpu.VMEM((48,), jnp.int32),
        pltpu.SemaphoreType.DMA,
    ],
)
def _segmax(h_hbm, src_hbm, dst_hbm, est_hbm, out_hbm,
            idx_v, dst_v, rows_v, acc_v, est_v, sem):
    wid = lax.axis_index("s") * 2 + lax.axis_index("c")
    lo = wid * ROWS_W

    pltpu.sync_copy(est_hbm, est_v)
    ev = est_v[pl.ds(wid, 16)]
    e0 = ev[0]
    e1 = ev[1] + 8

    neg = jnp.full((16,), -jnp.inf, jnp.float32)

    def init_row(r, carry):
        for cc in range(D // 16):
            acc_v[r, pl.ds(cc * 16, 16)] = neg
        return carry
    lax.fori_loop(0, ROWS_W, init_row, 0)

    nchunks = (e1 - e0 + CHUNK - 1) // CHUNK

    def chunk_body(g, carry):
        off = pl.multiple_of(e0 + g * CHUNK, 8)
        pltpu.sync_copy(src_hbm.at[pl.ds(off, CHUNK)], idx_v)
        pltpu.sync_copy(dst_hbm.at[pl.ds(off, CHUNK)], dst_v)
        pltpu.async_copy(h_hbm.at[idx_v], rows_v, sem).wait()

        def grp_body(g16, c2):
            base = g16 * 16
            dv = dst_v[pl.ds(base, 16)] - lo
            dc = jnp.where((dv >= 0) & (dv < ROWS_W), dv,
                           jnp.full((16,), ROWS_W, jnp.int32))
            for l in range(16):
                d = dc[l]
                for cc in range(D // 16):
                    sl = pl.ds(cc * 16, 16)
                    acc_v[d, sl] = jnp.maximum(acc_v[d, sl],
                                               rows_v[base + l, sl])
            return c2
        lax.fori_loop(0, CHUNK // 16, grp_body, 0)
        return carry
    lax.fori_loop(0, nchunks, chunk_body, 0)

    def fin_row(r, carry):
        for cc in range(D // 16):
            sl = pl.ds(cc * 16, 16)
            v = acc_v[r, sl]
            acc_v[r, sl] = jnp.where(v == -jnp.inf,
                                     jnp.zeros((16,), jnp.float32), v)
        return carry
    lax.fori_loop(0, ROWS_W, fin_row, 0)

    pltpu.sync_copy(acc_v.at[pl.ds(0, ROWS_W)], out_hbm.at[pl.ds(lo, ROWS_W)])


BLK = 1000


def _lin_body(agg_ref, h_ref, wr_ref, wt_ref, b_ref, o_ref):
    dn = (((1,), (1,)), ((), ()))
    o_ref[...] = jax.nn.relu(
        lax.dot_general(agg_ref[...], wr_ref[...], dn,
                        preferred_element_type=jnp.float32)
        + lax.dot_general(h_ref[...], wt_ref[...], dn,
                          preferred_element_type=jnp.float32)
        + b_ref[...])


_lin = pl.pallas_call(
    _lin_body,
    grid=(N // BLK,),
    in_specs=[
        pl.BlockSpec((BLK, D), lambda i: (i, 0)),
        pl.BlockSpec((BLK, D), lambda i: (i, 0)),
        pl.BlockSpec((D, D), lambda i: (0, 0)),
        pl.BlockSpec((D, D), lambda i: (0, 0)),
        pl.BlockSpec((1, D), lambda i: (0, 0)),
    ],
    out_specs=pl.BlockSpec((BLK, D), lambda i: (i, 0)),
    out_shape=jax.ShapeDtypeStruct((N, D), jnp.float32),
)


def kernel(x, edge_index, W1_rel, b1_rel, W1_root, W_rel, b_rel, W_root):
    src = edge_index[0].astype(jnp.int32)
    dst = edge_index[1].astype(jnp.int32)

    order = jnp.argsort(dst)
    dst_s = jnp.concatenate(
        [dst[order], jnp.full((EPAD,), N_PAD, jnp.int32)])
    src_s = jnp.concatenate([src[order], jnp.zeros((EPAD,), jnp.int32)])
    bounds = jnp.searchsorted(
        dst_s[:E], (jnp.arange(NW + 1) * ROWS_W).astype(jnp.int32)
    ).astype(jnp.int32)
    est = jnp.concatenate(
        [(bounds // 8) * 8, jnp.zeros((48 - (NW + 1),), jnp.int32)])

    h = x
    for wr, b, wt in ((W1_rel, b1_rel, W1_root),
                      (W_rel, b_rel, W_root),
                      (W_rel, b_rel, W_root)):
        agg = _segmax(h, src_s, dst_s, est)
        h = _lin(agg, h, wr, wt, b.reshape(1, D))
    return h


# speedup vs baseline: 3.2061x; 1.3135x over previous
import functools

import jax
import jax.numpy as jnp
from jax import lax
from jax.experimental import pallas as pl
from jax.experimental.pallas import tpu as pltpu
from jax.experimental.pallas import tpu_sc as plsc

N = 10000
E = 320000
D = 128
NW = 32
ROWS_W = 320
N_PAD = NW * ROWS_W
CHUNK = 128
EPAD = 256
NV = D // 16
NBUF = 4

_sc_mesh = plsc.VectorSubcoreMesh(core_axis_name="c", subcore_axis_name="s")


@functools.partial(
    pl.kernel,
    out_type=jax.ShapeDtypeStruct((N_PAD, D), jnp.float32),
    mesh=_sc_mesh,
    scratch_types=[
        pltpu.VMEM((NBUF, CHUNK), jnp.int32),
        pltpu.VMEM((NBUF, CHUNK), jnp.int32),
        pltpu.VMEM((NBUF, CHUNK, D), jnp.float32),
        pltpu.VMEM((ROWS_W + 1, D), jnp.float32),
        pltpu.VMEM((48,), jnp.int32),
        pltpu.SemaphoreType.DMA,
        pltpu.SemaphoreType.DMA,
        pltpu.SemaphoreType.DMA,
        pltpu.SemaphoreType.DMA,
        pltpu.SemaphoreType.DMA,
        pltpu.SemaphoreType.DMA,
        pltpu.SemaphoreType.DMA,
        pltpu.SemaphoreType.DMA,
    ],
)
def _segmax(h_hbm, src_hbm, dst_hbm, est_hbm, out_hbm,
            idx_v, dst_v, rows_v, acc_v, est_v,
            sr0, sr1, sr2, sr3, si0, si1, si2, si3):
    sem_r = (sr0, sr1, sr2, sr3)
    sem_i = (si0, si1, si2, si3)
    wid = lax.axis_index("s") * 2 + lax.axis_index("c")
    lo = wid * ROWS_W

    pltpu.sync_copy(est_hbm, est_v)
    ev = est_v[pl.ds(wid, 16)]
    e0 = ev[0]
    e1 = ev[1] + 8

    neg = jnp.full((16,), -jnp.inf, jnp.float32)

    def init_row(r, carry):
        for cc in range(NV):
            acc_v[r, pl.ds(cc * 16, 16)] = neg
        return carry
    lax.fori_loop(0, ROWS_W + 1, init_row, 0)

    nchunks = (e1 - e0 + CHUNK - 1) // CHUNK

    def stage(c, j):
        @pl.when(c < nchunks)
        def _():
            off = pl.multiple_of(e0 + c * CHUNK, 8)
            pltpu.async_copy(src_hbm.at[pl.ds(off, CHUNK)], idx_v.at[j],
                             sem_i[j])
            pltpu.async_copy(dst_hbm.at[pl.ds(off, CHUNK)], dst_v.at[j],
                             sem_i[j])

    def fire(c, j):
        @pl.when(c < nchunks)
        def _():
            pltpu.make_async_copy(src_hbm.at[pl.ds(0, CHUNK)], idx_v.at[j],
                                  sem_i[j]).wait()
            pltpu.make_async_copy(dst_hbm.at[pl.ds(0, CHUNK)], dst_v.at[j],
                                  sem_i[j]).wait()
            pltpu.async_copy(h_hbm.at[idx_v.at[j]], rows_v.at[j], sem_r[j])

    for j in range(NBUF):
        stage(j, j)
    for j in range(2):
        fire(j, j)

    def process(c, j):
        @pl.when(c < nchunks)
        def _():
            pltpu.make_async_copy(out_hbm.at[pl.ds(0, CHUNK)], rows_v.at[j],
                                  sem_r[j]).wait()

            def grp_body(g16, c2):
                base = g16 * 16
                dv = dst_v[j, pl.ds(base, 16)] - lo
                dc = jnp.where((dv >= 0) & (dv < ROWS_W), dv,
                               jnp.full((16,), ROWS_W, jnp.int32))
                for l in range(16):
                    d = dc[l]
                    for cc in range(NV):
                        sl = pl.ds(cc * 16, 16)
                        acc_v[d, sl] = jnp.maximum(
                            acc_v[d, sl], rows_v[j, base + l, sl])
                return c2
            lax.fori_loop(0, CHUNK // 16, grp_body, 0)

    def quad_body(q, carry):
        for j in range(NBUF):
            c = q * NBUF + j
            process(c, j)
            stage(c + NBUF, j)
            fire(c + 2, (j + 2) % NBUF)
        return carry

    nquads = (nchunks + NBUF - 1) // NBUF
    lax.fori_loop(0, nquads, quad_body, 0)

    def fin_row(r, carry):
        for cc in range(NV):
            sl = pl.ds(cc * 16, 16)
            v = acc_v[r, sl]
            acc_v[r, sl] = jnp.where(v == -jnp.inf,
                                     jnp.zeros((16,), jnp.float32), v)
        return carry
    lax.fori_loop(0, ROWS_W, fin_row, 0)

    pltpu.sync_copy(acc_v.at[pl.ds(0, ROWS_W)], out_hbm.at[pl.ds(lo, ROWS_W)])


BLK = 1000


def _lin_body(agg_ref, h_ref, wr_ref, wt_ref, b_ref, o_ref):
    dn = (((1,), (1,)), ((), ()))
    o_ref[...] = jax.nn.relu(
        lax.dot_general(agg_ref[...], wr_ref[...], dn,
                        preferred_element_type=jnp.float32)
        + lax.dot_general(h_ref[...], wt_ref[...], dn,
                          preferred_element_type=jnp.float32)
        + b_ref[...])


_lin = pl.pallas_call(
    _lin_body,
    grid=(N // BLK,),
    in_specs=[
        pl.BlockSpec((BLK, D), lambda i: (i, 0)),
        pl.BlockSpec((BLK, D), lambda i: (i, 0)),
        pl.BlockSpec((D, D), lambda i: (0, 0)),
        pl.BlockSpec((D, D), lambda i: (0, 0)),
        pl.BlockSpec((1, D), lambda i: (0, 0)),
    ],
    out_specs=pl.BlockSpec((BLK, D), lambda i: (i, 0)),
    out_shape=jax.ShapeDtypeStruct((N, D), jnp.float32),
)


def kernel(x, edge_index, W1_rel, b1_rel, W1_root, W_rel, b_rel, W_root):
    src = edge_index[0].astype(jnp.int32)
    dst = edge_index[1].astype(jnp.int32)

    order = jnp.argsort(dst)
    dst_s = jnp.concatenate(
        [dst[order], jnp.full((EPAD,), N_PAD, jnp.int32)])
    src_s = jnp.concatenate([src[order], jnp.zeros((EPAD,), jnp.int32)])
    bounds = jnp.searchsorted(
        dst_s[:E], (jnp.arange(NW + 1) * ROWS_W).astype(jnp.int32)
    ).astype(jnp.int32)
    est = jnp.concatenate(
        [(bounds // 8) * 8, jnp.zeros((48 - (NW + 1),), jnp.int32)])

    h = x
    for wr, b, wt in ((W1_rel, b1_rel, W1_root),
                      (W_rel, b_rel, W_root),
                      (W_rel, b_rel, W_root)):
        agg = _segmax(h, src_s, dst_s, est)
        h = _lin(agg, h, wr, wt, b.reshape(1, D))
    return h
